# Initial kernel scaffold; baseline (speedup 1.0000x reference)
import jax
import jax.numpy as jnp
from jax.experimental import pallas as pl


def kernel(token_ids, weight):
    raise NotImplementedError("write your pallas kernel here")


# speedup vs baseline: 1.3003x; 1.3003x over previous
import functools

import jax
import jax.numpy as jnp
from jax import lax
from jax.experimental import pallas as pl
from jax.experimental.pallas import tpu as pltpu
from jax.experimental.pallas import tpu_sc as plsc

NUM_EMB = 1_000_000
DIM = 32

NC = 2
NS = 16
NW = NC * NS

B_TOTAL = 16384 * 50
PER_W = B_TOTAL // NW
CHUNK = 128
N_CHUNKS = PER_W // CHUNK
K = 10
NG = N_CHUNKS // K


def _body(idx_hbm, table_hbm, out_hbm, idx_v, rows_v, gsem):
    wid = lax.axis_index("s") * NC + lax.axis_index("c")

    pltpu.sync_copy(idx_hbm.at[wid], idx_v)

    def group(g, carry):
        copies = []
        for j in range(K):
            copies.append(
                pltpu.async_copy(
                    table_hbm.at[idx_v.at[g * K + j]],
                    rows_v.at[j],
                    gsem,
                )
            )
        for c in copies:
            c.wait()
        pltpu.sync_copy(rows_v, out_hbm.at[wid, pl.ds(g * K, K)])
        return carry

    lax.fori_loop(0, NG, group, 0, unroll=False)


@functools.partial(jax.jit, static_argnames=())
def kernel(token_ids, weight):
    idx = token_ids.astype(jnp.int32).reshape(NW, N_CHUNKS, CHUNK)
    mesh = plsc.VectorSubcoreMesh(core_axis_name="c", subcore_axis_name="s")
    fn = pl.kernel(
        _body,
        mesh=mesh,
        out_type=jax.ShapeDtypeStruct((NW, N_CHUNKS, CHUNK, DIM), jnp.float32),
        scratch_types=[
            pltpu.VMEM((N_CHUNKS, CHUNK), jnp.int32),
            pltpu.VMEM((K, CHUNK, DIM), jnp.float32),
            pltpu.SemaphoreType.DMA,
        ],
        compiler_params=pltpu.CompilerParams(use_tc_tiling_on_sc=False),
    )
    out = fn(idx, weight)
    return out.reshape(16384, 50, DIM)


# speedup vs baseline: 1.7674x; 1.3593x over previous
import jax
import jax.numpy as jnp
from jax import lax
from jax.experimental import pallas as pl
from jax.experimental.pallas import tpu as pltpu
from jax.experimental.pallas import tpu_sc as plsc

NUM_EMB = 1_000_000
DIM = 32

NC = 2
NS = 16
NW = NC * NS

ROWS = 16384
COLS = 50
PER_W = ROWS // NW
K = 16
NG = PER_W // K


def _body(idx_hbm, table_hbm, out_hbm, idx_v, rows_v, gsem):
    wid = lax.axis_index("s") * NC + lax.axis_index("c")
    row0 = wid * PER_W

    pltpu.sync_copy(idx_hbm.at[pl.ds(row0, PER_W)], idx_v)

    def group(g, carry):
        copies = []
        for j in range(K):
            copies.append(
                pltpu.async_copy(
                    table_hbm.at[idx_v.at[g * K + j]],
                    rows_v.at[j],
                    gsem,
                )
            )
        for c in copies:
            c.wait()
        pltpu.sync_copy(rows_v, out_hbm.at[pl.ds(row0 + g * K, K)])
        return carry

    lax.fori_loop(0, NG, group, 0, unroll=False)


def kernel(token_ids, weight):
    w_lin = lax.optimization_barrier(weight.reshape(-1))
    w_row = w_lin.reshape(NUM_EMB, DIM)
    mesh = plsc.VectorSubcoreMesh(core_axis_name="c", subcore_axis_name="s")
    fn = pl.kernel(
        _body,
        mesh=mesh,
        out_type=jax.ShapeDtypeStruct((ROWS, COLS, DIM), jnp.float32),
        scratch_types=[
            pltpu.VMEM((PER_W, COLS), jnp.int32),
            pltpu.VMEM((K, COLS, DIM), jnp.float32),
            pltpu.SemaphoreType.DMA,
        ],
        compiler_params=pltpu.CompilerParams(use_tc_tiling_on_sc=False),
    )
    return fn(token_ids, w_row)
